# baseline (speedup 1.0000x reference)
import functools

import numpy as np
import jax
import jax.numpy as jnp
from jax import lax
from jax.experimental import pallas as pl
from jax.experimental.pallas import tpu as pltpu
from jax.experimental.pallas import tpu_sc as plsc

_LANES = 16
_NB = 2


def _sc_argmax_body(bpw, s_chunk, c, n, a_hbm, out_hbm, a_v0, a_v1, o_v0, o_v1,
                    isem0, isem1, osem0, osem1):
    ncores = 2
    wid = lax.axis_index("s") * ncores + lax.axis_index("c")
    nchunk = bpw // s_chunk
    f = c * n
    a_bufs = (a_v0, a_v1)
    o_bufs = (o_v0, o_v1)
    isems = (isem0, isem1)
    osems = (osem0, osem1)

    def in_copy(i, b):
        base = wid * bpw + i * s_chunk
        return pltpu.make_async_copy(
            a_hbm.at[pl.ds(base, s_chunk)], a_bufs[b], isems[b]
        )

    def out_copy(i, b):
        base = wid * bpw + i * s_chunk
        return pltpu.make_async_copy(
            o_bufs[b], out_hbm.at[pl.ds(base, s_chunk)], osems[b]
        )

    def compute(b):
        a_v = a_bufs[b]
        o_v = o_bufs[b]

        def sample_body(s, carry):
            for h in range(n // _LANES):
                off0 = h * _LANES
                m = a_v[s, pl.ds(off0, _LANES)]
                idx = jnp.zeros((_LANES,), jnp.int32)
                for ci in range(1, c):
                    v = a_v[s, pl.ds(ci * n + off0, _LANES)]
                    gt = v > m
                    idx = jnp.where(gt, jnp.full((_LANES,), ci, jnp.int32), idx)
                    m = jnp.maximum(m, v)
                o_v[s, pl.ds(off0, _LANES)] = idx
            return carry

        lax.fori_loop(0, s_chunk, sample_body, 0)

    for b in range(_NB):
        in_copy(b, b).start()
    for i in range(nchunk):
        b = i % _NB
        in_copy(i, b).wait()
        if i >= _NB:
            out_copy(i - _NB, b).wait()
        compute(b)
        out_copy(i, b).start()
        if i + _NB < nchunk:
            in_copy(i + _NB, b).start()
    for i in range(nchunk - _NB, nchunk):
        out_copy(i, i % _NB).wait()


def _sc_argmax(a2d, n, interpret=False):
    b, f = a2d.shape
    c = f // n
    nw = 32
    bpw = b // nw
    s_chunk = min(32, bpw)
    mesh = plsc.VectorSubcoreMesh(
        core_axis_name="c", subcore_axis_name="s", num_cores=2, num_subcores=16
    )
    body = functools.partial(_sc_argmax_body, bpw, s_chunk, c, n)
    return pl.kernel(
        body,
        out_type=jax.ShapeDtypeStruct((b, n), jnp.int32),
        mesh=mesh,
        scratch_types=[
            pltpu.VMEM((s_chunk, f), jnp.float32),
            pltpu.VMEM((s_chunk, f), jnp.float32),
            pltpu.VMEM((s_chunk, n), jnp.int32),
            pltpu.VMEM((s_chunk, n), jnp.int32),
            pltpu.SemaphoreType.DMA,
            pltpu.SemaphoreType.DMA,
            pltpu.SemaphoreType.DMA,
            pltpu.SemaphoreType.DMA,
        ],
        interpret=interpret,
    )(a2d)


def _expansion_matrix(n, c):
    return jnp.asarray(np.repeat(np.eye(n, dtype=np.float32), c, axis=1))


def _tc_out1_kernel(r_ref, e_ref, o1_ref):
    rf = r_ref[...].astype(jnp.float32)
    o1_ref[...] = jnp.dot(
        rf, e_ref[...], preferred_element_type=jnp.float32
    ).astype(jnp.int32)


def _tc_out1(r, c, interpret=False):
    b, n = r.shape
    f = c * n
    bb = min(1024, b)
    return pl.pallas_call(
        _tc_out1_kernel,
        interpret=interpret,
        grid=(b // bb,),
        in_specs=[
            pl.BlockSpec((bb, n), lambda i: (i, 0)),
            pl.BlockSpec((n, f), lambda i: (0, 0)),
        ],
        out_specs=pl.BlockSpec((bb, f), lambda i: (i, 0)),
        out_shape=jax.ShapeDtypeStruct((b, f), jnp.int32),
    )(r, _expansion_matrix(n, c))


def _tc_out0_kernel(c, bb, nh, idx1_ref, idx2_ref, r_ref, e_ref, o0_ref):
    f = e_ref.shape[1]
    in_first_half = pl.program_id(0) < nh
    xv = jnp.where(in_first_half, idx1_ref[...], idx2_ref[...])
    rf = r_ref[...].astype(jnp.float32)
    xf = xv.astype(jnp.float32)
    e = e_ref[...]
    rep_r = jnp.dot(rf, e, preferred_element_type=jnp.float32)
    rep_i = jnp.dot(xf, e, preferred_element_type=jnp.float32)
    cpat = (lax.broadcasted_iota(jnp.int32, (bb, f), 1) % c).astype(jnp.float32)
    o0_ref[...] = jnp.where(rep_i == cpat, rep_r, 0.0).astype(jnp.int32)


def _tc_out0(idx1, idx2, r, c, interpret=False):
    b, n = r.shape
    f = c * n
    bb = min(1024, b)
    nh = (b // bb) // 2
    return pl.pallas_call(
        functools.partial(_tc_out0_kernel, c, bb, nh),
        interpret=interpret,
        grid=(b // bb,),
        in_specs=[
            pl.BlockSpec((bb, n), lambda i: (jnp.minimum(i, nh - 1), 0)),
            pl.BlockSpec((bb, n), lambda i: (jnp.maximum(i - nh, 0), 0)),
            pl.BlockSpec((bb, n), lambda i: (i, 0)),
            pl.BlockSpec((n, f), lambda i: (0, 0)),
        ],
        out_specs=pl.BlockSpec((bb, f), lambda i: (i, 0)),
        out_shape=jax.ShapeDtypeStruct((b, f), jnp.int32),
    )(idx1, idx2, r, _expansion_matrix(n, c))


def kernel(ig_activations, sc_routing_matrix_curr_level):
    b, c, n = ig_activations.shape
    r = sc_routing_matrix_curr_level
    bh = b // 2
    a2d_1 = ig_activations[:bh].reshape(bh, c * n)
    a2d_2 = ig_activations[bh:].reshape(bh, c * n)
    idx1 = _sc_argmax(a2d_1, n)
    idx2 = _sc_argmax(a2d_2, n)
    out1 = _tc_out1(r, c)
    out0 = _tc_out0(idx1, idx2, r, c)
    return out0, out1


# speedup vs baseline: 1.2814x; 1.2814x over previous
import functools

import numpy as np
import jax
import jax.numpy as jnp
from jax import lax
from jax.experimental import pallas as pl
from jax.experimental.pallas import tpu as pltpu
from jax.experimental.pallas import tpu_sc as plsc

_LANES = 16
_NB = 2


def _sc_argmax_body(bpw, s_chunk, c, n, a_hbm, out_hbm, a_v0, a_v1, o_v0, o_v1,
                    isem0, isem1, osem0, osem1):
    ncores = 2
    wid = lax.axis_index("s") * ncores + lax.axis_index("c")
    nchunk = bpw // s_chunk
    f = c * n
    a_bufs = (a_v0, a_v1)
    o_bufs = (o_v0, o_v1)
    isems = (isem0, isem1)
    osems = (osem0, osem1)

    def in_copy(i, b):
        base = wid * bpw + i * s_chunk
        return pltpu.make_async_copy(
            a_hbm.at[pl.ds(base, s_chunk)], a_bufs[b], isems[b]
        )

    def out_copy(i, b):
        base = wid * bpw + i * s_chunk
        return pltpu.make_async_copy(
            o_bufs[b], out_hbm.at[pl.ds(base, s_chunk)], osems[b]
        )

    def compute(b):
        a_v = a_bufs[b]
        o_v = o_bufs[b]

        def sample_body(s, carry):
            for h in range(n // _LANES):
                off0 = h * _LANES
                m = a_v[s, pl.ds(off0, _LANES)]
                idx = jnp.zeros((_LANES,), jnp.int32)
                for ci in range(1, c):
                    v = a_v[s, pl.ds(ci * n + off0, _LANES)]
                    gt = v > m
                    idx = jnp.where(gt, jnp.full((_LANES,), ci, jnp.int32), idx)
                    m = jnp.maximum(m, v)
                o_v[s, pl.ds(off0, _LANES)] = idx
            return carry

        lax.fori_loop(0, s_chunk, sample_body, 0)

    for b in range(_NB):
        in_copy(b, b).start()
    for i in range(nchunk):
        b = i % _NB
        in_copy(i, b).wait()
        if i >= _NB:
            out_copy(i - _NB, b).wait()
        compute(b)
        out_copy(i, b).start()
        if i + _NB < nchunk:
            in_copy(i + _NB, b).start()
    for i in range(nchunk - _NB, nchunk):
        out_copy(i, i % _NB).wait()


def _sc_argmax(a2d, n, interpret=False):
    b, f = a2d.shape
    c = f // n
    nw = 32
    bpw = b // nw
    s_chunk = min(32, bpw)
    mesh = plsc.VectorSubcoreMesh(
        core_axis_name="c", subcore_axis_name="s", num_cores=2, num_subcores=16
    )
    body = functools.partial(_sc_argmax_body, bpw, s_chunk, c, n)
    return pl.kernel(
        body,
        out_type=jax.ShapeDtypeStruct((b, n), jnp.int32),
        mesh=mesh,
        scratch_types=[
            pltpu.VMEM((s_chunk, f), jnp.float32),
            pltpu.VMEM((s_chunk, f), jnp.float32),
            pltpu.VMEM((s_chunk, n), jnp.int32),
            pltpu.VMEM((s_chunk, n), jnp.int32),
            pltpu.SemaphoreType.DMA,
            pltpu.SemaphoreType.DMA,
            pltpu.SemaphoreType.DMA,
            pltpu.SemaphoreType.DMA,
        ],
        interpret=interpret,
    )(a2d)


def _expansion_matrix(n, c):
    return jnp.asarray(np.repeat(np.eye(n, dtype=np.float32), c, axis=1))


def _tc_expand_kernel(c, bb, idx_ref, r_ref, e_ref, o0_ref, o1_ref):
    f = e_ref.shape[1]
    rf = r_ref[...].astype(jnp.float32)
    xf = idx_ref[...].astype(jnp.float32)
    e = e_ref[...]
    rep_r = jnp.dot(rf, e, preferred_element_type=jnp.float32)
    rep_i = jnp.dot(xf, e, preferred_element_type=jnp.float32)
    cpat = (lax.broadcasted_iota(jnp.int32, (bb, f), 1) % c).astype(jnp.float32)
    o0_ref[...] = jnp.where(rep_i == cpat, rep_r, 0.0).astype(jnp.int32)
    o1_ref[...] = rep_r.astype(jnp.int32)


def _tc_expand(idx, r, c, bb=2048, interpret=False):
    b, n = r.shape
    f = c * n
    bb = min(bb, b)
    out = jax.ShapeDtypeStruct((b, f), jnp.int32)
    return pl.pallas_call(
        functools.partial(_tc_expand_kernel, c, bb),
        interpret=interpret,
        grid=(b // bb,),
        in_specs=[
            pl.BlockSpec((bb, n), lambda i: (i, 0)),
            pl.BlockSpec((bb, n), lambda i: (i, 0)),
            pl.BlockSpec((n, f), lambda i: (0, 0)),
        ],
        out_specs=(
            pl.BlockSpec((bb, f), lambda i: (i, 0)),
            pl.BlockSpec((bb, f), lambda i: (i, 0)),
        ),
        out_shape=(out, out),
    )(idx, r, _expansion_matrix(n, c))


def kernel(ig_activations, sc_routing_matrix_curr_level):
    b, c, n = ig_activations.shape
    r = sc_routing_matrix_curr_level
    a2d = ig_activations.reshape(b, c * n)
    idx = _sc_argmax(a2d, n)
    out0, out1 = _tc_expand(idx, r, c)
    return out0, out1
